# Initial kernel scaffold; baseline (speedup 1.0000x reference)
import jax
import jax.numpy as jnp
from jax.experimental import pallas as pl


def kernel(x, W_enc, W_dec, b_enc, b_dec):
    raise NotImplementedError("write your pallas kernel here")


# speedup vs baseline: 13.6248x; 13.6248x over previous
import jax
import jax.numpy as jnp
from jax import lax
from jax.experimental import pallas as pl
from jax.experimental.pallas import tpu as pltpu

D_IN = 2304
D_SAE = 18432
TOPK = 100
HALF = D_SAE // 2
BATCH = 1024
ROWS2 = 2 * BATCH

ENC_TILE = 512
DEC_TILE = 512
SIM_TILE = 512
K2_ROWS = 128

_PREC = lax.Precision.DEFAULT


def _k1_body(x_ref, w_ref, b_ref, out_ref):
    out_ref[...] = (
        jnp.dot(x_ref[...], w_ref[...], preferred_element_type=jnp.float32,
                precision=_PREC)
        + b_ref[...]
    )


def _encode_pre(x2, w_enc, b_enc2):
    grid = (D_SAE // ENC_TILE,)
    return pl.pallas_call(
        _k1_body,
        grid=grid,
        in_specs=[
            pl.BlockSpec((ROWS2, D_IN), lambda j: (0, 0)),
            pl.BlockSpec((D_IN, ENC_TILE), lambda j: (0, j)),
            pl.BlockSpec((1, ENC_TILE), lambda j: (0, j)),
        ],
        out_specs=pl.BlockSpec((ROWS2, ENC_TILE), lambda j: (0, j)),
        out_shape=jax.ShapeDtypeStruct((ROWS2, D_SAE), jnp.float32),
    )(x2, w_enc, b_enc2)


def _k2_body(pre_ref, z_ref, nh2_ref):
    p = pre_ref[...]
    bits = lax.bitcast_convert_type(p, jnp.int32)
    bitsp = jnp.where(p > 0.0, bits, 0)
    t = jnp.zeros((K2_ROWS, 1), jnp.int32)
    for b in range(30, -1, -1):
        cand = t + (1 << b)
        cnt = jnp.sum((bitsp >= cand).astype(jnp.int32), axis=1, keepdims=True)
        t = jnp.where(cnt >= TOPK, cand, t)
    thr = jnp.maximum(t, 1)
    z = jnp.where(bitsp >= thr, p, 0.0)
    z_ref[...] = z
    col = lax.broadcasted_iota(jnp.int32, (K2_ROWS, D_SAE), 1)
    zh = jnp.where(col < HALF, z, 0.0)
    nh2_ref[...] = jnp.sum(zh * zh, axis=1, keepdims=True)


def _topk_mask(pre):
    grid = (ROWS2 // K2_ROWS,)
    return pl.pallas_call(
        _k2_body,
        grid=grid,
        in_specs=[pl.BlockSpec((K2_ROWS, D_SAE), lambda i: (i, 0))],
        out_specs=[
            pl.BlockSpec((K2_ROWS, D_SAE), lambda i: (i, 0)),
            pl.BlockSpec((K2_ROWS, 1), lambda i: (i, 0)),
        ],
        out_shape=[
            jax.ShapeDtypeStruct((ROWS2, D_SAE), jnp.float32),
            jax.ShapeDtypeStruct((ROWS2, 1), jnp.float32),
        ],
    )(pre)


def _k3_body(z_ref, w_ref, xc_ref, bd_ref, xhat_ref, lhi_ref, lfull_ref,
             acchi_ref):
    c = pl.program_id(0)
    nhi = HALF // DEC_TILE

    @pl.when(c == 0)
    def _init():
        xhat_ref[...] = jnp.zeros_like(xhat_ref)
        acchi_ref[...] = jnp.zeros_like(acchi_ref)

    prod = jnp.dot(z_ref[...], w_ref[...], preferred_element_type=jnp.float32,
                   precision=_PREC)
    xhat_ref[...] += prod

    @pl.when(c < nhi)
    def _hi():
        acchi_ref[...] += prod

    @pl.when(c == (D_SAE // DEC_TILE) - 1)
    def _fin():
        bd = bd_ref[...]
        xc = xc_ref[...]
        full = xhat_ref[...] + bd
        xhat_ref[...] = full
        d = full - xc
        ef = jnp.sum(jnp.sum(d * d, axis=1, keepdims=True), axis=0,
                     keepdims=True)
        hi = acchi_ref[...] + bd
        dh = hi - xc
        eh = jnp.sum(jnp.sum(dh * dh, axis=1, keepdims=True), axis=0,
                     keepdims=True)
        denom = float(BATCH * D_IN)
        lfull_ref[...] = ef / denom
        lhi_ref[...] = eh / denom


def _decode_losses(z, w_dec, x_cur, b_dec2):
    grid = (D_SAE // DEC_TILE,)
    return pl.pallas_call(
        _k3_body,
        grid=grid,
        in_specs=[
            pl.BlockSpec((BATCH, DEC_TILE), lambda c: (0, c)),
            pl.BlockSpec((DEC_TILE, D_IN), lambda c: (c, 0)),
            pl.BlockSpec((BATCH, D_IN), lambda c: (0, 0)),
            pl.BlockSpec((1, D_IN), lambda c: (0, 0)),
        ],
        out_specs=[
            pl.BlockSpec((BATCH, D_IN), lambda c: (0, 0)),
            pl.BlockSpec((1, 1), lambda c: (0, 0)),
            pl.BlockSpec((1, 1), lambda c: (0, 0)),
        ],
        out_shape=[
            jax.ShapeDtypeStruct((BATCH, D_IN), jnp.float32),
            jax.ShapeDtypeStruct((1, 1), jnp.float32),
            jax.ShapeDtypeStruct((1, 1), jnp.float32),
        ],
        scratch_shapes=[pltpu.VMEM((BATCH, D_IN), jnp.float32)],
    )(z, w_dec, x_cur, b_dec2)


def _k4_body(za_ref, zb_ref, na2_ref, nb2_ref, out_ref, sim_ref):
    j = pl.program_id(0)

    @pl.when(j == 0)
    def _init():
        sim_ref[...] = jnp.zeros_like(sim_ref)

    na = jnp.maximum(jnp.sqrt(na2_ref[...]), 1e-8)
    nb = jnp.maximum(jnp.sqrt(nb2_ref[...]), 1e-8)
    zan = za_ref[...] / na
    zbn = zb_ref[...] / nb
    sim_ref[...] += lax.dot_general(
        zan, zbn, (((1,), (1,)), ((), ())),
        preferred_element_type=jnp.float32, precision=_PREC)

    @pl.when(j == (HALF // SIM_TILE) - 1)
    def _fin():
        s = sim_ref[...]
        rmax = jnp.max(s, axis=1, keepdims=True)
        lse_r = rmax + jnp.log(jnp.sum(jnp.exp(s - rmax), axis=1,
                                       keepdims=True))
        cmax = jnp.max(s, axis=0, keepdims=True)
        lse_c = cmax + jnp.log(jnp.sum(jnp.exp(s - cmax), axis=0,
                                       keepdims=True))
        ii = lax.broadcasted_iota(jnp.int32, (BATCH, BATCH), 0)
        jj = lax.broadcasted_iota(jnp.int32, (BATCH, BATCH), 1)
        dsum = jnp.sum(jnp.sum(jnp.where(ii == jj, s, 0.0), axis=1,
                               keepdims=True), axis=0, keepdims=True)
        sr = jnp.sum(jnp.sum(lse_r, axis=1, keepdims=True), axis=0,
                     keepdims=True)
        sc = jnp.sum(jnp.sum(lse_c, axis=1, keepdims=True), axis=0,
                     keepdims=True)
        out_ref[...] = (-dsum + 0.5 * (sr + sc)) / float(BATCH)


def _info_nce_loss(z, nh2):
    grid = (HALF // SIM_TILE,)
    return pl.pallas_call(
        _k4_body,
        grid=grid,
        in_specs=[
            pl.BlockSpec((BATCH, SIM_TILE), lambda j: (0, j)),
            pl.BlockSpec((BATCH, SIM_TILE), lambda j: (1, j)),
            pl.BlockSpec((BATCH, 1), lambda j: (0, 0)),
            pl.BlockSpec((BATCH, 1), lambda j: (1, 0)),
        ],
        out_specs=pl.BlockSpec((1, 1), lambda j: (0, 0)),
        out_shape=jax.ShapeDtypeStruct((1, 1), jnp.float32),
        scratch_shapes=[pltpu.VMEM((BATCH, BATCH), jnp.float32)],
    )(z, z, nh2, nh2)


def kernel(x, W_enc, W_dec, b_enc, b_dec):
    x_cur = x[:, 1, :]
    x_prev = x[:, 0, :]
    x2 = jnp.concatenate([x_cur, x_prev], axis=0)
    b_enc2 = b_enc.reshape(1, D_SAE)
    b_dec2 = b_dec.reshape(1, D_IN)

    pre = _encode_pre(x2, W_enc, b_enc2)
    z, nh2 = _topk_mask(pre)
    x_hat, l_hi, l_full = _decode_losses(z[:BATCH], W_dec, x_cur, b_dec2)
    l_contr = _info_nce_loss(z, nh2)

    total = (l_hi[0, 0] + l_full[0, 0]) + l_contr[0, 0]
    return (total, x_hat, z[:BATCH])
